# baseline (speedup 1.0000x reference)
import functools

import jax
import jax.numpy as jnp
from jax import lax
from jax.experimental import pallas as pl
from jax.experimental.pallas import tpu as pltpu
from jax.experimental.pallas import tpu_sc as plsc

_N = 10000
_D = 256
_E = 160000
_K = 4
_ALPHA = 0.9
_BETA = 0.1

_NC = 2
_NS = 16
_HALF = _D // _NC
_NPAD = 10240
_RPT = _NPAD // _NS
_SUB = 128
_NSUB = _RPT // _SUB
_CH = 128
_NCH = 80
_GRP = 16
_NGRP = _NCH // _GRP
_EPT = _CH * _NCH
_EPAD = _EPT * _NS
_ZROW = _NPAD - _SUB


def _gate_body(x_ref, w1_ref, b1_ref, w2_ref, b2_ref, o_ref):
    h = jnp.dot(x_ref[...], w1_ref[...], preferred_element_type=jnp.float32)
    h = jnp.maximum(h + b1_ref[...], 0.0)
    lg = jnp.dot(h, w2_ref[...], preferred_element_type=jnp.float32)
    lg = lg + b2_ref[...]
    m = jnp.max(lg, axis=-1, keepdims=True)
    e = jnp.exp(lg - m)
    o_ref[...] = e / jnp.sum(e, axis=-1, keepdims=True)


def _hop_gate(x_pad, w1, b1, w2, b2):
    blk = 1024
    grid = (_NPAD // blk,)
    w2p = jnp.zeros((_D, 128), jnp.float32).at[:, :_K].set(w2)
    b2p = jnp.full((1, 128), -1e30, jnp.float32).at[0, :_K].set(b2)
    return pl.pallas_call(
        _gate_body,
        grid=grid,
        in_specs=[
            pl.BlockSpec((blk, _D), lambda i: (i, 0)),
            pl.BlockSpec((_D, _D), lambda i: (0, 0)),
            pl.BlockSpec((1, _D), lambda i: (0, 0)),
            pl.BlockSpec((_D, 128), lambda i: (0, 0)),
            pl.BlockSpec((1, 128), lambda i: (0, 0)),
        ],
        out_specs=pl.BlockSpec((blk, 128), lambda i: (i, 0)),
        out_shape=jax.ShapeDtypeStruct((_NPAD, 128), jnp.float32),
    )(x_pad, w1, b1.reshape(1, _D), w2p, b2p)


def _combine_body(hp1, hp2, hp3, hp4, hw_ref, dinv_ref, o_ref):
    acc = hp1[0, 0] * hw_ref[:, 0:1]
    acc = acc + hp2[0, 0] * hw_ref[:, 1:2]
    acc = acc + hp3[0, 0] * hw_ref[:, 2:3]
    acc = acc + hp4[0, 0] * hw_ref[:, 3:4]
    o_ref[0] = acc / dinv_ref[0][:, 0:1]


def _combine(hp, hw_full, dinv):
    blk = 1024
    grid = (_NPAD // blk, _NC)
    hp_specs = [
        pl.BlockSpec((1, 1, blk, _HALF), lambda i, c, k=k: (k, c, i, 0))
        for k in range(1, _K + 1)
    ]
    return pl.pallas_call(
        _combine_body,
        grid=grid,
        in_specs=hp_specs + [
            pl.BlockSpec((blk, 128), lambda i, c: (i, 0)),
            pl.BlockSpec((1, blk, 16), lambda i, c: (c, i, 0)),
        ],
        out_specs=pl.BlockSpec((1, blk, _HALF), lambda i, c: (c, i, 0)),
        out_shape=jax.ShapeDtypeStruct((_NC, _NPAD, _HALF), jnp.float32),
    )(hp, hp, hp, hp, hw_full, dinv)


def _rsqrt16(x):
    y = 1.0 / (0.0625 * x + 4.0)
    for _ in range(12):
        y = y * (1.5 - (0.5 * x) * (y * y))
    return y


def _sc_body(xs_hbm, src_hbm, dst_hbm, hp_hbm, dinv_hbm,
             agg, gbufa, gbufb, srcblk, dstblk, dinvbuf,
             gsa, gsb, ssa, ssb):
    c = lax.axis_index("c")
    s = lax.axis_index("s")
    own = s * _RPT

    xs_c = xs_hbm.at[c]
    dinv_c = dinv_hbm.at[c]
    zeros_hbm = xs_c.at[pl.ds(_ZROW, _SUB)]

    one16 = jnp.ones((16,), jnp.float32)

    def _fill_ones(r, _):
        for i in range(8):
            gbufa[r, pl.ds(16 * i, 16)] = one16
        return 0
    lax.fori_loop(0, _CH, _fill_ones, 0)

    def _zero_sub(t, _):
        pltpu.sync_copy(zeros_hbm, agg.at[pl.ds(own + _SUB * t, _SUB)])
        return 0
    lax.fori_loop(0, _NSUB, _zero_sub, 0)
    plsc.subcore_barrier()

    def _deg_grp(g, _):
        pltpu.sync_copy(dst_hbm.at[s, pl.ds(_GRP * g, _GRP)], dstblk)

        def _deg_chunk(j, __):
            pltpu.sync_copy(gbufa, agg.at[dstblk.at[j]], add=True)
            return 0
        lax.fori_loop(0, _GRP, _deg_chunk, 0)
        return 0
    lax.fori_loop(0, _NGRP, _deg_grp, 0)
    plsc.subcore_barrier()

    def _init_sub(t, _):
        base = own + _SUB * t
        pltpu.sync_copy(agg.at[pl.ds(base, _SUB)], gbufa)
        pltpu.sync_copy(zeros_hbm, agg.at[pl.ds(base, _SUB)])
        pltpu.sync_copy(xs_c.at[pl.ds(base, _SUB)], gbufb)

        def _hp0_blk(q, __):
            def _hp0_row(r, ___):
                rr = 32 * q + r
                dv = _rsqrt16(jnp.maximum(gbufa[rr, pl.ds(0, 16)], 1.0))
                dinvbuf[r, :] = dv
                for i in range(8):
                    sl = pl.ds(16 * i, 16)
                    gbufb[rr, sl] = dv * gbufb[rr, sl]
                return 0
            lax.fori_loop(0, 32, _hp0_row, 0)
            pltpu.sync_copy(dinvbuf, dinv_c.at[pl.ds(base + 32 * q, 32)])
            return 0
        lax.fori_loop(0, _SUB // 32, _hp0_blk, 0)

        pltpu.sync_copy(gbufb, hp_hbm.at[0, c, pl.ds(base, _SUB)])
        return 0
    lax.fori_loop(0, _NSUB, _init_sub, 0)
    plsc.subcore_barrier()

    for k in range(1, _K + 1):
        hp_prev = hp_hbm.at[k - 1, c]
        hp_next = hp_hbm.at[k, c]

        def _edge_grp(g, _, hp_prev=hp_prev):
            pltpu.sync_copy(src_hbm.at[s, pl.ds(_GRP * g, _GRP)], srcblk)
            pltpu.sync_copy(dst_hbm.at[s, pl.ds(_GRP * g, _GRP)], dstblk)

            def _ga(j):
                pltpu.async_copy(hp_prev.at[srcblk.at[j]], gbufa, gsa)

            def _gb(j):
                pltpu.async_copy(hp_prev.at[srcblk.at[j]], gbufb, gsb)

            def _sa(j):
                pltpu.async_copy(gbufa, agg.at[dstblk.at[j]], ssa, add=True)

            def _sb(j):
                pltpu.async_copy(gbufb, agg.at[dstblk.at[j]], ssb, add=True)

            def _wga(j):
                pltpu.make_async_copy(hp_prev.at[srcblk.at[j]], gbufa,
                                      gsa).wait()

            def _wgb(j):
                pltpu.make_async_copy(hp_prev.at[srcblk.at[j]], gbufb,
                                      gsb).wait()

            def _wsa(j):
                pltpu.make_async_copy(gbufa, agg.at[dstblk.at[j]],
                                      ssa).wait()

            def _wsb(j):
                pltpu.make_async_copy(gbufb, agg.at[dstblk.at[j]],
                                      ssb).wait()

            _ga(0)
            _gb(1)

            def _pipe(m, __):
                j0 = 2 * m
                j1 = 2 * m + 1
                _wga(j0)
                _sa(j0)
                _wgb(j1)
                _sb(j1)
                _wsa(j0)
                _ga(j0 + 2)
                _wsb(j1)
                _gb(j1 + 2)
                return 0
            lax.fori_loop(0, _GRP // 2 - 2, _pipe, 0)

            _wga(_GRP - 4)
            _sa(_GRP - 4)
            _wgb(_GRP - 3)
            _sb(_GRP - 3)
            _wsa(_GRP - 4)
            _ga(_GRP - 2)
            _wsb(_GRP - 3)
            _gb(_GRP - 1)
            _wga(_GRP - 2)
            _sa(_GRP - 2)
            _wgb(_GRP - 1)
            _sb(_GRP - 1)
            _wsa(_GRP - 2)
            _wsb(_GRP - 1)
            return 0
        lax.fori_loop(0, _NGRP, _edge_grp, 0)
        plsc.subcore_barrier()

        def _upd_sub(t, _, k=k, hp_next=hp_next):
            base = own + _SUB * t
            pltpu.sync_copy(agg.at[pl.ds(base, _SUB)], gbufa)
            if k < _K:
                pltpu.sync_copy(zeros_hbm, agg.at[pl.ds(base, _SUB)])
            pltpu.sync_copy(xs_c.at[pl.ds(base, _SUB)], gbufb)

            def _upd_blk(q, __):
                pltpu.sync_copy(dinv_c.at[pl.ds(base + 32 * q, 32)], dinvbuf)

                def _upd_row(r, ___):
                    rr = 32 * q + r
                    dv = dinvbuf[r, :]
                    for i in range(8):
                        sl = pl.ds(16 * i, 16)
                        h = (_ALPHA * (dv * gbufa[rr, sl])
                             + _BETA * gbufb[rr, sl])
                        gbufb[rr, sl] = dv * h
                    return 0
                lax.fori_loop(0, 32, _upd_row, 0)
                return 0
            lax.fori_loop(0, _SUB // 32, _upd_blk, 0)

            pltpu.sync_copy(gbufb, hp_next.at[pl.ds(base, _SUB)])
            return 0
        lax.fori_loop(0, _NSUB, _upd_sub, 0)
        plsc.subcore_barrier()


def _sc_propagate(xs, srcz, dstz):
    mesh = plsc.VectorSubcoreMesh(
        core_axis_name="c", subcore_axis_name="s",
        num_cores=_NC, num_subcores=_NS)
    fn = pl.kernel(
        _sc_body,
        out_type=(
            jax.ShapeDtypeStruct((_K + 1, _NC, _NPAD, _HALF), jnp.float32),
            jax.ShapeDtypeStruct((_NC, _NPAD, 16), jnp.float32),
        ),
        mesh=mesh,
        scratch_types=[
            pltpu.VMEM_SHARED((_NPAD, _HALF), jnp.float32),
            pltpu.VMEM((_CH, _HALF), jnp.float32),
            pltpu.VMEM((_CH, _HALF), jnp.float32),
            pltpu.VMEM((_GRP, _CH), jnp.int32),
            pltpu.VMEM((_GRP, _CH), jnp.int32),
            pltpu.VMEM((32, 16), jnp.float32),
            pltpu.SemaphoreType.DMA,
            pltpu.SemaphoreType.DMA,
            pltpu.SemaphoreType.DMA,
            pltpu.SemaphoreType.DMA,
        ],
    )
    return fn(xs, srcz, dstz)


def kernel(x, edge_index, w1, b1, w2, b2):
    x_pad = jnp.zeros((_NPAD, _D), jnp.float32).at[:_N].set(x)
    hw_full = _hop_gate(x_pad, w1, b1, w2, b2)

    xs = x_pad.reshape(_NPAD, _NC, _HALF).transpose(1, 0, 2)

    src = edge_index[0].astype(jnp.int32)
    dst = edge_index[1].astype(jnp.int32)
    npad_e = _EPAD - _E
    src_p = jnp.concatenate([src, jnp.zeros((npad_e,), jnp.int32)])
    dst_p = jnp.concatenate(
        [dst, _N + (jnp.arange(npad_e, dtype=jnp.int32) % (_NPAD - _N))])
    srcz = src_p.reshape(_NS, _NCH, _CH)
    dstz = dst_p.reshape(_NS, _NCH, _CH)

    hp, dinv = _sc_propagate(xs, srcz, dstz)
    out_split = _combine(hp, hw_full, dinv)
    out = out_split.transpose(1, 0, 2).reshape(_NPAD, _D)[:_N]
    return out, hw_full[:_N, :_K]


# speedup vs baseline: 1.0331x; 1.0331x over previous
import functools

import jax
import jax.numpy as jnp
from jax import lax
from jax.experimental import pallas as pl
from jax.experimental.pallas import tpu as pltpu
from jax.experimental.pallas import tpu_sc as plsc

_N = 10000
_D = 256
_E = 160000
_K = 4
_ALPHA = 0.9
_BETA = 0.1

_NC = 2
_NS = 16
_HALF = _D // _NC
_NPAD = 10240
_RPT = _NPAD // _NS
_SUB = 128
_NSUB = _RPT // _SUB
_CH = 128
_NCH = 80
_GRP = 16
_NGRP = _NCH // _GRP
_EPT = _CH * _NCH
_EPAD = _EPT * _NS
_ZROW = _NPAD - _SUB


def _gate_body(x_ref, w1_ref, b1_ref, w2_ref, b2_ref, o_ref):
    h = jnp.dot(x_ref[...], w1_ref[...], preferred_element_type=jnp.float32)
    h = jnp.maximum(h + b1_ref[...], 0.0)
    lg = jnp.dot(h, w2_ref[...], preferred_element_type=jnp.float32)
    lg = lg + b2_ref[...]
    m = jnp.max(lg, axis=-1, keepdims=True)
    e = jnp.exp(lg - m)
    o_ref[...] = e / jnp.sum(e, axis=-1, keepdims=True)


def _hop_gate(x_pad, w1, b1, w2, b2):
    blk = 1024
    grid = (_NPAD // blk,)
    w2p = jnp.zeros((_D, 128), jnp.float32).at[:, :_K].set(w2)
    b2p = jnp.full((1, 128), -1e30, jnp.float32).at[0, :_K].set(b2)
    return pl.pallas_call(
        _gate_body,
        grid=grid,
        in_specs=[
            pl.BlockSpec((blk, _D), lambda i: (i, 0)),
            pl.BlockSpec((_D, _D), lambda i: (0, 0)),
            pl.BlockSpec((1, _D), lambda i: (0, 0)),
            pl.BlockSpec((_D, 128), lambda i: (0, 0)),
            pl.BlockSpec((1, 128), lambda i: (0, 0)),
        ],
        out_specs=pl.BlockSpec((blk, 128), lambda i: (i, 0)),
        out_shape=jax.ShapeDtypeStruct((_NPAD, 128), jnp.float32),
    )(x_pad, w1, b1.reshape(1, _D), w2p, b2p)


def _combine_body(hp1, hp2, hp3, hp4, hw_ref, dinv_ref, o_ref):
    acc = hp1[0, 0] * hw_ref[:, 0:1]
    acc = acc + hp2[0, 0] * hw_ref[:, 1:2]
    acc = acc + hp3[0, 0] * hw_ref[:, 2:3]
    acc = acc + hp4[0, 0] * hw_ref[:, 3:4]
    o_ref[0] = acc / dinv_ref[0][:, 0:1]


def _combine(hp, hw_full, dinv):
    blk = 1024
    grid = (_NPAD // blk, _NC)
    hp_specs = [
        pl.BlockSpec((1, 1, blk, _HALF), lambda i, c, k=k: (k, c, i, 0))
        for k in range(1, _K + 1)
    ]
    return pl.pallas_call(
        _combine_body,
        grid=grid,
        in_specs=hp_specs + [
            pl.BlockSpec((blk, 128), lambda i, c: (i, 0)),
            pl.BlockSpec((1, blk, 16), lambda i, c: (c, i, 0)),
        ],
        out_specs=pl.BlockSpec((1, blk, _HALF), lambda i, c: (c, i, 0)),
        out_shape=jax.ShapeDtypeStruct((_NC, _NPAD, _HALF), jnp.float32),
    )(hp, hp, hp, hp, hw_full, dinv)


def _rsqrt16(x):
    y = 1.0 / (0.0625 * x + 4.0)
    for _ in range(12):
        y = y * (1.5 - (0.5 * x) * (y * y))
    return y


def _sc_body(xs_hbm, src_hbm, dst_hbm, hp_hbm, dinv_hbm,
             agg, gbufa, gbufb, srcblk, dstblk, dinvbuf,
             gsa, gsb, ssa, ssb):
    c = lax.axis_index("c")
    s = lax.axis_index("s")
    own = s * _RPT

    xs_c = xs_hbm.at[c]
    dinv_c = dinv_hbm.at[c]
    zeros_hbm = xs_c.at[pl.ds(_ZROW, _SUB)]

    one16 = jnp.ones((16,), jnp.float32)

    def _fill_ones(r, _):
        for i in range(8):
            gbufa[r, pl.ds(16 * i, 16)] = one16
        return 0
    lax.fori_loop(0, _CH, _fill_ones, 0)

    def _zero_sub(t, _):
        pltpu.sync_copy(zeros_hbm, agg.at[pl.ds(own + _SUB * t, _SUB)])
        return 0
    lax.fori_loop(0, _NSUB, _zero_sub, 0)
    plsc.subcore_barrier()

    def _deg_grp(g, _):
        pltpu.sync_copy(dst_hbm.at[s, pl.ds(_GRP * g, _GRP)], dstblk)

        def _deg_chunk(j, __):
            pltpu.sync_copy(gbufa, agg.at[dstblk.at[j]], add=True)
            return 0
        lax.fori_loop(0, _GRP, _deg_chunk, 0)
        return 0
    lax.fori_loop(0, _NGRP, _deg_grp, 0)
    plsc.subcore_barrier()

    def _init_sub(t, _):
        base = own + _SUB * t
        pltpu.sync_copy(agg.at[pl.ds(base, _SUB)], gbufa)
        pltpu.sync_copy(zeros_hbm, agg.at[pl.ds(base, _SUB)])
        pltpu.sync_copy(xs_c.at[pl.ds(base, _SUB)], gbufb)

        def _hp0_blk(q, __):
            def _hp0_row(r, ___):
                rr = 64 * q + r
                dv = _rsqrt16(jnp.maximum(gbufa[rr, pl.ds(0, 16)], 1.0))
                dinvbuf[r, :] = dv
                for i in range(8):
                    sl = pl.ds(16 * i, 16)
                    gbufb[rr, sl] = dv * gbufb[rr, sl]
                return 0
            lax.fori_loop(0, 64, _hp0_row, 0)
            pltpu.sync_copy(dinvbuf, dinv_c.at[pl.ds(base + 64 * q, 64)])
            return 0
        lax.fori_loop(0, _SUB // 64, _hp0_blk, 0)

        pltpu.sync_copy(gbufb, hp_hbm.at[0, c, pl.ds(base, _SUB)])
        return 0
    lax.fori_loop(0, _NSUB, _init_sub, 0)
    plsc.subcore_barrier()

    for k in range(1, _K + 1):
        hp_prev = hp_hbm.at[k - 1, c]
        hp_next = hp_hbm.at[k, c]

        def _edge_grp(g, _, hp_prev=hp_prev):
            pltpu.sync_copy(src_hbm.at[s, pl.ds(_GRP * g, _GRP)], srcblk)
            pltpu.sync_copy(dst_hbm.at[s, pl.ds(_GRP * g, _GRP)], dstblk)

            pltpu.async_copy(hp_prev.at[srcblk.at[0]], gbufa, gsa).wait()
            pltpu.async_copy(gbufa, agg.at[dstblk.at[0]], ssa, add=True)
            pltpu.async_copy(hp_prev.at[srcblk.at[1]], gbufb, gsb)

            def _pipe(m, __):
                jb = 2 * m + 1
                ja = 2 * m + 2
                pltpu.make_async_copy(hp_prev.at[srcblk.at[jb]], gbufb,
                                      gsb).wait()
                pltpu.async_copy(gbufb, agg.at[dstblk.at[jb]], ssb, add=True)
                pltpu.make_async_copy(gbufa, agg.at[dstblk.at[ja]],
                                      ssa).wait()
                pltpu.async_copy(hp_prev.at[srcblk.at[ja]], gbufa, gsa)
                pltpu.make_async_copy(hp_prev.at[srcblk.at[ja]], gbufa,
                                      gsa).wait()
                pltpu.async_copy(gbufa, agg.at[dstblk.at[ja]], ssa, add=True)
                pltpu.make_async_copy(gbufb, agg.at[dstblk.at[jb]],
                                      ssb).wait()
                pltpu.async_copy(hp_prev.at[srcblk.at[jb + 2]], gbufb, gsb)
                return 0
            lax.fori_loop(0, _GRP // 2 - 1, _pipe, 0)

            pltpu.make_async_copy(hp_prev.at[srcblk.at[_GRP - 1]], gbufb,
                                  gsb).wait()
            pltpu.async_copy(gbufb, agg.at[dstblk.at[_GRP - 1]], ssb, add=True)
            pltpu.make_async_copy(gbufa, agg.at[dstblk.at[_GRP - 2]],
                                  ssa).wait()
            pltpu.make_async_copy(gbufb, agg.at[dstblk.at[_GRP - 1]],
                                  ssb).wait()
            return 0
        lax.fori_loop(0, _NGRP, _edge_grp, 0)
        plsc.subcore_barrier()

        def _upd_sub(t, _, k=k, hp_next=hp_next):
            base = own + _SUB * t
            pltpu.sync_copy(agg.at[pl.ds(base, _SUB)], gbufa)
            if k < _K:
                pltpu.async_copy(zeros_hbm, agg.at[pl.ds(base, _SUB)], ssa)
            pltpu.make_async_copy(gbufb, hp_next.at[pl.ds(own, _SUB)],
                                  ssb).wait()
            pltpu.sync_copy(xs_c.at[pl.ds(base, _SUB)], gbufb)

            def _upd_blk(q, __):
                pltpu.sync_copy(dinv_c.at[pl.ds(base + 64 * q, 64)], dinvbuf)

                def _upd_row(r, ___):
                    rr = 64 * q + r
                    dv = dinvbuf[r, :]
                    for i in range(8):
                        sl = pl.ds(16 * i, 16)
                        h = (_ALPHA * (dv * gbufa[rr, sl])
                             + _BETA * gbufb[rr, sl])
                        gbufb[rr, sl] = dv * h
                    return 0
                lax.fori_loop(0, 64, _upd_row, 0)
                return 0
            lax.fori_loop(0, _SUB // 64, _upd_blk, 0)

            pltpu.async_copy(gbufb, hp_next.at[pl.ds(base, _SUB)], ssb)
            return 0
        pltpu.async_copy(gbufb, hp_next.at[pl.ds(own, _SUB)], ssb)
        lax.fori_loop(0, _NSUB, _upd_sub, 0)
        pltpu.make_async_copy(gbufb, hp_next.at[pl.ds(own, _SUB)], ssb).wait()
        if k < _K:
            def _drain_z(t, _, k=k, hp_next=hp_next):
                pltpu.make_async_copy(zeros_hbm, agg.at[pl.ds(own, _SUB)],
                                      ssa).wait()
                return 0
            lax.fori_loop(0, _NSUB, _drain_z, 0)
        plsc.subcore_barrier()


def _sc_propagate(xs, srcz, dstz):
    mesh = plsc.VectorSubcoreMesh(
        core_axis_name="c", subcore_axis_name="s",
        num_cores=_NC, num_subcores=_NS)
    fn = pl.kernel(
        _sc_body,
        out_type=(
            jax.ShapeDtypeStruct((_K + 1, _NC, _NPAD, _HALF), jnp.float32),
            jax.ShapeDtypeStruct((_NC, _NPAD, 16), jnp.float32),
        ),
        mesh=mesh,
        scratch_types=[
            pltpu.VMEM_SHARED((_NPAD, _HALF), jnp.float32),
            pltpu.VMEM((_CH, _HALF), jnp.float32),
            pltpu.VMEM((_CH, _HALF), jnp.float32),
            pltpu.VMEM((_GRP, _CH), jnp.int32),
            pltpu.VMEM((_GRP, _CH), jnp.int32),
            pltpu.VMEM((64, 16), jnp.float32),
            pltpu.SemaphoreType.DMA,
            pltpu.SemaphoreType.DMA,
            pltpu.SemaphoreType.DMA,
            pltpu.SemaphoreType.DMA,
        ],
    )
    return fn(xs, srcz, dstz)


def kernel(x, edge_index, w1, b1, w2, b2):
    x_pad = jnp.zeros((_NPAD, _D), jnp.float32).at[:_N].set(x)
    hw_full = _hop_gate(x_pad, w1, b1, w2, b2)

    xs = x_pad.reshape(_NPAD, _NC, _HALF).transpose(1, 0, 2)

    src = edge_index[0].astype(jnp.int32)
    dst = edge_index[1].astype(jnp.int32)
    npad_e = _EPAD - _E
    src_p = jnp.concatenate([src, jnp.zeros((npad_e,), jnp.int32)])
    dst_p = jnp.concatenate(
        [dst, _N + (jnp.arange(npad_e, dtype=jnp.int32) % (_NPAD - _N))])
    srcz = src_p.reshape(_NS, _NCH, _CH)
    dstz = dst_p.reshape(_NS, _NCH, _CH)

    hp, dinv = _sc_propagate(xs, srcz, dstz)
    out_split = _combine(hp, hw_full, dinv)
    out = out_split.transpose(1, 0, 2).reshape(_NPAD, _D)[:_N]
    return out, hw_full[:_N, :_K]


# speedup vs baseline: 1.0336x; 1.0005x over previous
import jax
import jax.numpy as jnp
from jax import lax
from jax.experimental import pallas as pl
from jax.experimental.pallas import tpu as pltpu
from jax.experimental.pallas import tpu_sc as plsc

_N = 10000
_D = 256
_E = 160000
_K = 4
_ALPHA = 0.9
_BETA = 0.1

_NC = 2
_NS = 16
_HALF = _D // _NC
_NPAD = 10240
_RPT = _NPAD // _NS
_SUB = 128
_NSUB = _RPT // _SUB
_CH = 128
_NCH = 80
_GRP = 16
_NGRP = _NCH // _GRP
_EPT = _CH * _NCH
_EPAD = _EPT * _NS
_ZROW = _NPAD - _SUB


def _gate_body(x_ref, w1_ref, b1_ref, w2_ref, b2_ref, o_ref):
    h = jnp.dot(x_ref[...], w1_ref[...], preferred_element_type=jnp.float32)
    h = jnp.maximum(h + b1_ref[...], 0.0)
    lg = jnp.dot(h, w2_ref[...], preferred_element_type=jnp.float32)
    lg = lg + b2_ref[...]
    m = jnp.max(lg, axis=-1, keepdims=True)
    e = jnp.exp(lg - m)
    o_ref[...] = e / jnp.sum(e, axis=-1, keepdims=True)


def _hop_gate(x_pad, w1, b1, w2, b2):
    blk = 1024
    grid = (_NPAD // blk,)
    w2p = jnp.zeros((_D, 128), jnp.float32).at[:, :_K].set(w2)
    b2p = jnp.full((1, 128), -1e30, jnp.float32).at[0, :_K].set(b2)
    return pl.pallas_call(
        _gate_body,
        grid=grid,
        in_specs=[
            pl.BlockSpec((blk, _D), lambda i: (i, 0)),
            pl.BlockSpec((_D, _D), lambda i: (0, 0)),
            pl.BlockSpec((1, _D), lambda i: (0, 0)),
            pl.BlockSpec((_D, 128), lambda i: (0, 0)),
            pl.BlockSpec((1, 128), lambda i: (0, 0)),
        ],
        out_specs=pl.BlockSpec((blk, 128), lambda i: (i, 0)),
        out_shape=jax.ShapeDtypeStruct((_NPAD, 128), jnp.float32),
    )(x_pad, w1, b1.reshape(1, _D), w2p, b2p)


def _combine_body(hp1, hp2, hp3, hp4, hw_ref, dinv_ref, o_ref):
    acc = hp1[0, 0] * hw_ref[:, 0:1]
    acc = acc + hp2[0, 0] * hw_ref[:, 1:2]
    acc = acc + hp3[0, 0] * hw_ref[:, 2:3]
    acc = acc + hp4[0, 0] * hw_ref[:, 3:4]
    o_ref[0] = acc / dinv_ref[0][:, 0:1]


def _combine(hp, hw_full, dinv):
    blk = 1024
    grid = (_NPAD // blk, _NC)
    hp_specs = [
        pl.BlockSpec((1, 1, blk, _HALF), lambda i, c, k=k: (k, c, i, 0))
        for k in range(1, _K + 1)
    ]
    return pl.pallas_call(
        _combine_body,
        grid=grid,
        in_specs=hp_specs + [
            pl.BlockSpec((blk, 128), lambda i, c: (i, 0)),
            pl.BlockSpec((1, blk, 16), lambda i, c: (c, i, 0)),
        ],
        out_specs=pl.BlockSpec((1, blk, _HALF), lambda i, c: (c, i, 0)),
        out_shape=jax.ShapeDtypeStruct((_NC, _NPAD, _HALF), jnp.float32),
    )(hp, hp, hp, hp, hw_full, dinv)


def _rsqrt16(x):
    y = 1.0 / (0.0625 * x + 4.0)
    for _ in range(12):
        y = y * (1.5 - (0.5 * x) * (y * y))
    return y


def _sc_body(xs_hbm, src_hbm, dst_hbm, hp_hbm, dinv_hbm,
             agg, gbufa, gbufb, srcblk, dstblk, dinvbuf,
             gsa, gsb, ssa, ssb):
    c = lax.axis_index("c")
    s = lax.axis_index("s")
    own = s * _RPT

    xs_c = xs_hbm.at[c]
    dinv_c = dinv_hbm.at[c]
    zeros_hbm = xs_c.at[pl.ds(_ZROW, _SUB)]

    one16 = jnp.ones((16,), jnp.float32)

    def _fill_ones(r, _):
        for i in range(8):
            gbufa[r, pl.ds(16 * i, 16)] = one16
        return 0
    lax.fori_loop(0, _CH, _fill_ones, 0)

    def _zero_sub(t, _):
        pltpu.sync_copy(zeros_hbm, agg.at[pl.ds(own + _SUB * t, _SUB)])
        return 0
    lax.fori_loop(0, _NSUB, _zero_sub, 0)
    plsc.subcore_barrier()

    def _deg_grp(g, _):
        pltpu.sync_copy(dst_hbm.at[s, pl.ds(_GRP * g, _GRP)], dstblk)

        def _deg_chunk(j, __):
            pltpu.sync_copy(gbufa, agg.at[dstblk.at[j]], add=True)
            return 0
        lax.fori_loop(0, _GRP, _deg_chunk, 0)
        return 0
    lax.fori_loop(0, _NGRP, _deg_grp, 0)
    plsc.subcore_barrier()

    def _init_sub(t, _):
        base = own + _SUB * t
        pltpu.sync_copy(agg.at[pl.ds(base, _SUB)], gbufa)
        pltpu.sync_copy(zeros_hbm, agg.at[pl.ds(base, _SUB)])
        pltpu.sync_copy(xs_c.at[pl.ds(base, _SUB)], gbufb)

        def _hp0_blk(q, __):
            def _hp0_row(r, ___):
                rr = 64 * q + r
                dv = _rsqrt16(jnp.maximum(gbufa[rr, pl.ds(0, 16)], 1.0))
                dinvbuf[r, :] = dv
                for i in range(8):
                    sl = pl.ds(16 * i, 16)
                    gbufb[rr, sl] = dv * gbufb[rr, sl]
                return 0
            lax.fori_loop(0, 64, _hp0_row, 0)
            pltpu.sync_copy(dinvbuf, dinv_c.at[pl.ds(base + 64 * q, 64)])
            return 0
        lax.fori_loop(0, _SUB // 64, _hp0_blk, 0)

        pltpu.sync_copy(gbufb, hp_hbm.at[0, c, pl.ds(base, _SUB)])
        return 0
    lax.fori_loop(0, _NSUB, _init_sub, 0)
    plsc.subcore_barrier()

    for k in range(1, _K + 1):
        hp_prev = hp_hbm.at[k - 1, c]
        hp_next = hp_hbm.at[k, c]

        def _edge_grp(g, _, hp_prev=hp_prev):
            pltpu.sync_copy(src_hbm.at[s, pl.ds(_GRP * g, _GRP)], srcblk)
            pltpu.sync_copy(dst_hbm.at[s, pl.ds(_GRP * g, _GRP)], dstblk)

            pltpu.async_copy(hp_prev.at[srcblk.at[0]], gbufa, gsa).wait()
            pltpu.async_copy(gbufa, agg.at[dstblk.at[0]], ssa, add=True)
            pltpu.async_copy(hp_prev.at[srcblk.at[1]], gbufb, gsb)

            def _pipe(m, __):
                jb = 2 * m + 1
                ja = 2 * m + 2
                pltpu.make_async_copy(hp_prev.at[srcblk.at[jb]], gbufb,
                                      gsb).wait()
                pltpu.async_copy(gbufb, agg.at[dstblk.at[jb]], ssb, add=True)
                pltpu.make_async_copy(gbufa, agg.at[dstblk.at[ja]],
                                      ssa).wait()
                pltpu.async_copy(hp_prev.at[srcblk.at[ja]], gbufa, gsa)
                pltpu.make_async_copy(hp_prev.at[srcblk.at[ja]], gbufa,
                                      gsa).wait()
                pltpu.async_copy(gbufa, agg.at[dstblk.at[ja]], ssa, add=True)
                pltpu.make_async_copy(gbufb, agg.at[dstblk.at[jb]],
                                      ssb).wait()
                pltpu.async_copy(hp_prev.at[srcblk.at[jb + 2]], gbufb, gsb)
                return 0
            lax.fori_loop(0, _GRP // 2 - 1, _pipe, 0)

            pltpu.make_async_copy(hp_prev.at[srcblk.at[_GRP - 1]], gbufb,
                                  gsb).wait()
            pltpu.async_copy(gbufb, agg.at[dstblk.at[_GRP - 1]], ssb, add=True)
            pltpu.make_async_copy(gbufa, agg.at[dstblk.at[_GRP - 2]],
                                  ssa).wait()
            pltpu.make_async_copy(gbufb, agg.at[dstblk.at[_GRP - 1]],
                                  ssb).wait()
            return 0
        lax.fori_loop(0, _NGRP, _edge_grp, 0)
        plsc.subcore_barrier()

        def _upd_sub(t, _, k=k, hp_next=hp_next):
            base = own + _SUB * t
            pltpu.sync_copy(agg.at[pl.ds(base, _SUB)], gbufa)
            if k < _K:
                pltpu.async_copy(zeros_hbm, agg.at[pl.ds(base, _SUB)], ssa)
            pltpu.make_async_copy(gbufb, hp_next.at[pl.ds(own, _SUB)],
                                  ssb).wait()
            pltpu.sync_copy(xs_c.at[pl.ds(base, _SUB)], gbufb)

            def _upd_blk(q, __):
                pltpu.sync_copy(dinv_c.at[pl.ds(base + 64 * q, 64)], dinvbuf)

                def _upd_row(r, ___):
                    rr = 64 * q + r
                    dv = dinvbuf[r, :]
                    for i in range(8):
                        sl = pl.ds(16 * i, 16)
                        h = (_ALPHA * (dv * gbufa[rr, sl])
                             + _BETA * gbufb[rr, sl])
                        gbufb[rr, sl] = dv * h
                    return 0
                lax.fori_loop(0, 64, _upd_row, 0)
                return 0
            lax.fori_loop(0, _SUB // 64, _upd_blk, 0)

            pltpu.async_copy(gbufb, hp_next.at[pl.ds(base, _SUB)], ssb)
            return 0
        pltpu.async_copy(gbufb, hp_next.at[pl.ds(own, _SUB)], ssb)
        lax.fori_loop(0, _NSUB, _upd_sub, 0)
        pltpu.make_async_copy(gbufb, hp_next.at[pl.ds(own, _SUB)], ssb).wait()
        if k < _K:
            def _drain_z(t, _, k=k, hp_next=hp_next):
                pltpu.make_async_copy(zeros_hbm, agg.at[pl.ds(own, _SUB)],
                                      ssa).wait()
                return 0
            lax.fori_loop(0, _NSUB, _drain_z, 0)
        plsc.subcore_barrier()


def _sc_propagate(xs, srcz, dstz):
    mesh = plsc.VectorSubcoreMesh(
        core_axis_name="c", subcore_axis_name="s",
        num_cores=_NC, num_subcores=_NS)
    fn = pl.kernel(
        _sc_body,
        out_type=(
            jax.ShapeDtypeStruct((_K + 1, _NC, _NPAD, _HALF), jnp.float32),
            jax.ShapeDtypeStruct((_NC, _NPAD, 16), jnp.float32),
        ),
        mesh=mesh,
        scratch_types=[
            pltpu.VMEM_SHARED((_NPAD, _HALF), jnp.float32),
            pltpu.VMEM((_CH, _HALF), jnp.float32),
            pltpu.VMEM((_CH, _HALF), jnp.float32),
            pltpu.VMEM((_GRP, _CH), jnp.int32),
            pltpu.VMEM((_GRP, _CH), jnp.int32),
            pltpu.VMEM((64, 16), jnp.float32),
            pltpu.SemaphoreType.DMA,
            pltpu.SemaphoreType.DMA,
            pltpu.SemaphoreType.DMA,
            pltpu.SemaphoreType.DMA,
        ],
    )
    return fn(xs, srcz, dstz)


def kernel(x, edge_index, w1, b1, w2, b2):
    x_pad = jnp.zeros((_NPAD, _D), jnp.float32).at[:_N].set(x)
    hw_full = _hop_gate(x_pad, w1, b1, w2, b2)

    xs = x_pad.reshape(_NPAD, _NC, _HALF).transpose(1, 0, 2)

    src = edge_index[0].astype(jnp.int32)
    dst = edge_index[1].astype(jnp.int32)
    npad_e = _EPAD - _E
    src_p = jnp.concatenate([src, jnp.zeros((npad_e,), jnp.int32)])
    dst_p = jnp.concatenate(
        [dst, _N + (jnp.arange(npad_e, dtype=jnp.int32) % (_NPAD - _N))])
    srcz = src_p.reshape(_NS, _NCH, _CH)
    dstz = dst_p.reshape(_NS, _NCH, _CH)

    hp, dinv = _sc_propagate(xs, srcz, dstz)
    out_split = _combine(hp, hw_full, dinv)
    out = out_split.transpose(1, 0, 2).reshape(_NPAD, _D)[:_N]
    return out, hw_full[:_N, :_K]
